# baseline (speedup 1.0000x reference)
import functools

import jax
import jax.numpy as jnp
from jax import lax
from jax.experimental import pallas as pl
from jax.experimental.pallas import tpu as pltpu
from jax.experimental.pallas import tpu_sc as plsc

EMB_DIM = 64
BATCH = 16384

NUM_CORES = 2
NUM_SUBCORES = 16
NUM_WORKERS = NUM_CORES * NUM_SUBCORES
B_PER_W = BATCH // NUM_WORKERS
CHUNK = 128
CHUNKS = B_PER_W // CHUNK
PIPE = 4


@functools.partial(
    pl.kernel,
    mesh=plsc.VectorSubcoreMesh(core_axis_name="c", subcore_axis_name="s"),
    out_type=jax.ShapeDtypeStruct((EMB_DIM, BATCH), jnp.float32),
    compiler_params=pltpu.CompilerParams(use_tc_tiling_on_sc=False),
    scratch_types=[
        pltpu.VMEM((CHUNKS, CHUNK), jnp.int32),
        pltpu.VMEM((EMB_DIM, B_PER_W), jnp.float32),
        pltpu.SemaphoreType.DMA,
    ],
)
def _sc_gather_t(idx_hbm, table_t_hbm, out_t_hbm, idx_v, cols_v, sem):
    wid = lax.axis_index("s") * NUM_CORES + lax.axis_index("c")
    pltpu.sync_copy(idx_hbm.at[wid], idx_v)

    pending = []
    for c in range(EMB_DIM):
        row = []
        for q in range(CHUNKS):
            row.append(
                pltpu.async_copy(
                    table_t_hbm.at[c].at[idx_v.at[q]],
                    cols_v.at[c, pl.ds(q * CHUNK, CHUNK)],
                    sem,
                )
            )
        pending.append(row)
        if c >= PIPE:
            for cp in pending[c - PIPE]:
                cp.wait()
    for row in pending[EMB_DIM - PIPE:]:
        for cp in row:
            cp.wait()

    pltpu.sync_copy(cols_v, out_t_hbm.at[:, pl.ds(wid * B_PER_W, B_PER_W)])


def kernel(x, table):
    idx = x.astype(jnp.int32).reshape(NUM_WORKERS, CHUNKS, CHUNK)
    out_t = _sc_gather_t(idx, table.T)
    return out_t.T


# speedup vs baseline: 12.5843x; 12.5843x over previous
import functools

import jax
import jax.numpy as jnp
from jax import lax
from jax.experimental import pallas as pl
from jax.experimental.pallas import tpu as pltpu
from jax.experimental.pallas import tpu_sc as plsc

EMB_DIM = 64
BATCH = 16384

NUM_CORES = 2
NUM_SUBCORES = 16
NUM_WORKERS = NUM_CORES * NUM_SUBCORES
B_PER_W = BATCH // NUM_WORKERS
LANES = 16
GROUPS = B_PER_W // LANES


@functools.partial(
    pl.kernel,
    mesh=plsc.VectorSubcoreMesh(core_axis_name="c", subcore_axis_name="s"),
    out_type=jax.ShapeDtypeStruct((BATCH, EMB_DIM), jnp.float32),
    compiler_params=pltpu.CompilerParams(needs_layout_passes=False),
    scratch_types=[
        pltpu.VMEM((GROUPS, LANES), jnp.int32),
        pltpu.VMEM((2, LANES, 8, EMB_DIM), jnp.float32),
        pltpu.VMEM((B_PER_W // 8, 8, EMB_DIM), jnp.float32),
        pltpu.SemaphoreType.DMA,
    ],
)
def _sc_gather(idx_hbm, table_hbm, out_hbm, idx_v, blk_v, stg_v, sem):
    wid = lax.axis_index("s") * NUM_CORES + lax.axis_index("c")
    base = wid * B_PER_W
    pltpu.sync_copy(idx_hbm.at[wid], idx_v)

    lane_ids = lax.iota(jnp.int32, LANES)

    def do_group(g, buf):
        v = idx_v[g]
        copies = []
        for f in range(LANES):
            i = v[f]
            blk = pl.multiple_of((i >> 3) * 8, 8)
            copies.append(
                pltpu.async_copy(
                    table_hbm.at[pl.ds(blk, 8), :],
                    blk_v.at[buf, f],
                    sem,
                )
            )
        for cp in copies:
            cp.wait()
        ri = v & 7
        for f in range(LANES):
            f_vec = jnp.full((LANES,), f, jnp.int32)
            ri_vec = jnp.full((LANES,), 1, jnp.int32) * ri[f]
            for k in range(EMB_DIM // LANES):
                c_vec = lane_ids + k * LANES
                vals = plsc.load_gather(
                    blk_v.at[buf], [f_vec, ri_vec, c_vec]
                )
                stg_v[2 * g + f // 8, f % 8, pl.ds(k * LANES, LANES)] = vals

    def body(g, _):
        pl.when(g % 2 == 0)(lambda: do_group(g, 0))
        pl.when(g % 2 == 1)(lambda: do_group(g, 1))
        return ()

    lax.fori_loop(0, GROUPS, body, (), unroll=1)

    out_copies = []
    for t in range(B_PER_W // 8):
        out_copies.append(
            pltpu.async_copy(
                stg_v.at[t], out_hbm.at[pl.ds(base + 8 * t, 8), :], sem
            )
        )
    for cp in out_copies:
        cp.wait()


def kernel(x, table):
    idx = x.astype(jnp.int32).reshape(NUM_WORKERS, GROUPS, LANES)
    return _sc_gather(idx, table)
